# Initial kernel scaffold; baseline (speedup 1.0000x reference)
import jax
import jax.numpy as jnp
from jax.experimental import pallas as pl


def kernel(x, edge_index, params):
    raise NotImplementedError("write your pallas kernel here")


# speedup vs baseline: 53.5662x; 53.5662x over previous
import functools

import jax
import jax.numpy as jnp
from jax import lax
from jax.experimental import pallas as pl
from jax.experimental.pallas import tpu as pltpu
from jax.experimental.pallas import tpu_sc as plsc

N = 10000
E = 320000
D = 128
H = 8
DH = 16
S = 32

NC = 2
NS = 16
NW = NC * NS
NP = 10240
CH = 128
NCHUNK = 80
EP = NW * NCHUNK * CH
RPT = NP // NS
NBLK = NP // 128
SPARE = NP - N

_MASKVAL = -1e30


def _deg_kernel(dst_hbm, out_hbm, didx_all, ones_v, out_sp):
    cid = lax.axis_index("c")
    sid = lax.axis_index("s")
    wid = sid * NC + cid

    pltpu.sync_copy(dst_hbm.at[wid], didx_all)

    @pl.loop(0, CH)
    def _z(i):
        for c in range(8):
            ones_v[i, pl.ds(c * 16, 16)] = jnp.zeros((16,), jnp.float32)

    for r in range(RPT // CH):
        pltpu.sync_copy(ones_v, out_sp.at[pl.ds(sid * RPT + r * CH, CH)])

    @pl.loop(0, CH)
    def _o(i):
        for c in range(8):
            ones_v[i, pl.ds(c * 16, 16)] = jnp.ones((16,), jnp.float32)

    plsc.subcore_barrier()

    @pl.loop(0, NCHUNK)
    def _main(j):
        pltpu.sync_copy(ones_v, out_sp.at[didx_all.at[j]], add=True)

    plsc.subcore_barrier()
    pltpu.sync_copy(out_sp.at[pl.ds(sid * RPT, RPT)],
                    out_hbm.at[cid, pl.ds(sid * RPT, RPT)])


@jax.jit
def _deg_call(dst_idx):
    return pl.kernel(
        _deg_kernel,
        out_type=jax.ShapeDtypeStruct((NC, NP, D), jnp.float32),
        mesh=plsc.VectorSubcoreMesh(core_axis_name="c", subcore_axis_name="s"),
        scratch_types=[
            pltpu.VMEM((NCHUNK, CH), jnp.int32),
            pltpu.VMEM((CH, D), jnp.float32),
            pltpu.VMEM_SHARED((NP, D), jnp.float32),
        ],
    )(dst_idx)


def _prop_kernel(ys_hbm, src_hbm, dst_hbm, out_hbm,
                 s0, d0, s1, d1, rows0, rows1, out_sp, g0, g1):
    cid = lax.axis_index("c")
    sid = lax.axis_index("s")
    wid = sid * NC + cid

    @pl.loop(0, CH)
    def _z(i):
        for c in range(8):
            rows0[i, pl.ds(c * 16, 16)] = jnp.zeros((16,), jnp.float32)

    for r in range(RPT // CH):
        pltpu.sync_copy(rows0, out_sp.at[pl.ds(sid * RPT + r * CH, CH)])
    plsc.subcore_barrier()

    pltpu.sync_copy(src_hbm.at[wid, 0], s0)
    pltpu.sync_copy(dst_hbm.at[wid, 0], d0)
    pltpu.async_copy(ys_hbm.at[s0], rows0, g0)
    pltpu.sync_copy(src_hbm.at[wid, 1], s1)
    pltpu.sync_copy(dst_hbm.at[wid, 1], d1)
    pltpu.async_copy(ys_hbm.at[s1], rows1, g1)

    @pl.loop(0, NCHUNK // 2)
    def _main(i):
        j0 = i * 2
        pltpu.make_async_copy(ys_hbm.at[s0], rows0, g0).wait()
        pltpu.sync_copy(rows0, out_sp.at[d0], add=True)

        @pl.when(i < NCHUNK // 2 - 1)
        def _pref0():
            pltpu.sync_copy(src_hbm.at[wid, j0 + 2], s0)
            pltpu.sync_copy(dst_hbm.at[wid, j0 + 2], d0)
            pltpu.async_copy(ys_hbm.at[s0], rows0, g0)

        pltpu.make_async_copy(ys_hbm.at[s1], rows1, g1).wait()
        pltpu.sync_copy(rows1, out_sp.at[d1], add=True)

        @pl.when(i < NCHUNK // 2 - 1)
        def _pref1():
            pltpu.sync_copy(src_hbm.at[wid, j0 + 3], s1)
            pltpu.sync_copy(dst_hbm.at[wid, j0 + 3], d1)
            pltpu.async_copy(ys_hbm.at[s1], rows1, g1)

    plsc.subcore_barrier()
    pltpu.sync_copy(out_sp.at[pl.ds(sid * RPT, RPT)],
                    out_hbm.at[cid, pl.ds(sid * RPT, RPT)])


@jax.jit
def _prop_call(ys, src_idx, dst_idx):
    return pl.kernel(
        _prop_kernel,
        out_type=jax.ShapeDtypeStruct((NC, NP, D), jnp.float32),
        mesh=plsc.VectorSubcoreMesh(core_axis_name="c", subcore_axis_name="s"),
        scratch_types=[
            pltpu.VMEM((CH,), jnp.int32),
            pltpu.VMEM((CH,), jnp.int32),
            pltpu.VMEM((CH,), jnp.int32),
            pltpu.VMEM((CH,), jnp.int32),
            pltpu.VMEM((CH, D), jnp.float32),
            pltpu.VMEM((CH, D), jnp.float32),
            pltpu.VMEM_SHARED((NP, D), jnp.float32),
            pltpu.SemaphoreType.DMA,
            pltpu.SemaphoreType.DMA,
        ],
    )(ys, src_idx, dst_idx)


def _tc0_kernel(deg0_ref, deg1_ref, x_ref, dinv_ref, ys_ref):
    i = pl.program_id(0)
    deg = deg0_ref[...] + deg1_ref[...]
    rows = i * 128 + lax.broadcasted_iota(jnp.int32, (128, D), 0)
    dinv_rep = jnp.where(rows < N, lax.rsqrt(deg + 1.0), 0.0)
    dinv_ref[...] = dinv_rep
    ys_ref[...] = x_ref[...] * dinv_rep


@jax.jit
def _tc0(deg_parts, x_pad):
    bd = lambda: pl.BlockSpec((128, D), lambda i: (i, 0))
    return pl.pallas_call(
        _tc0_kernel,
        grid=(NBLK,),
        in_specs=[bd(), bd(), bd()],
        out_specs=[bd(), bd()],
        out_shape=[jax.ShapeDtypeStruct((NP, D), jnp.float32),
                   jax.ShapeDtypeStruct((NP, D), jnp.float32)],
    )(deg_parts[0], deg_parts[1], x_pad)


def _tc1_kernel(u0_ref, u1_ref, ys_ref, dinv_ref, w_ref, b_ref, out_ref):
    dinv = dinv_ref[...]
    p = dinv * (u0_ref[...] + u1_ref[...] + ys_ref[...])
    h = jnp.maximum(jnp.dot(p, w_ref[...],
                            preferred_element_type=jnp.float32) + b_ref[...], 0.0)
    out_ref[...] = dinv * h


@jax.jit
def _tc1(u0, u1, ys, dinv_rep, w, b):
    bd = lambda: pl.BlockSpec((128, D), lambda i: (i, 0))
    full = lambda r: pl.BlockSpec((r, D), lambda i: (0, 0))
    return pl.pallas_call(
        _tc1_kernel,
        grid=(NBLK,),
        in_specs=[bd(), bd(), bd(), bd(), full(D), full(1)],
        out_specs=bd(),
        out_shape=jax.ShapeDtypeStruct((NP, D), jnp.float32),
    )(u0, u1, ys, dinv_rep, w, b.reshape(1, D))


def _head_mask(nrep):
    row_h = lax.broadcasted_iota(jnp.int32, (H * nrep, D), 0) // nrep
    col_h = lax.broadcasted_iota(jnp.int32, (H * nrep, D), 1) // DH
    return (row_h == col_h).astype(jnp.float32)


def _tc2_kernel(u0_ref, u1_ref, ys_ref, dinv_ref, wk_ref, bk_ref,
                wv_ref, bv_ref, seed_ref, qw_ref, pooled_ref,
                qbig, m_scr, s_scr, acc_scr):
    i = pl.program_id(0)

    @pl.when(i == 0)
    def _init():
        q = lax.dot_general(seed_ref[...], qw_ref[...],
                            (((1,), (1,)), ((), ())),
                            preferred_element_type=jnp.float32) * (DH ** -0.5)
        qbig[...] = jnp.tile(q, (H, 1)) * _head_mask(S)
        m_scr[...] = jnp.full((H * S, 128), _MASKVAL, jnp.float32)
        s_scr[...] = jnp.zeros((H * S, 128), jnp.float32)
        acc_scr[...] = jnp.zeros((H * S, D), jnp.float32)

    p3 = dinv_ref[...] * (u0_ref[...] + u1_ref[...] + ys_ref[...])
    k = jnp.dot(p3, wk_ref[...], preferred_element_type=jnp.float32) + bk_ref[...]
    v = jnp.dot(p3, wv_ref[...], preferred_element_type=jnp.float32) + bv_ref[...]

    logits = lax.dot_general(qbig[...], k, (((1,), (1,)), ((), ())),
                             preferred_element_type=jnp.float32)
    cols = i * 128 + lax.broadcasted_iota(jnp.int32, (H * S, 128), 1)
    logits = jnp.where(cols < N, logits, _MASKVAL)

    m_old = m_scr[...]
    m_new = jnp.maximum(m_old, jnp.max(logits, axis=-1, keepdims=True))
    alpha = jnp.exp(m_old - m_new)
    probs = jnp.exp(logits - m_new)
    m_scr[...] = jnp.broadcast_to(m_new[:, :1], (H * S, 128))
    s_scr[...] = s_scr[...] * alpha + jnp.sum(probs, axis=-1, keepdims=True)
    acc_scr[...] = acc_scr[...] * alpha[:, :1] + jnp.dot(
        probs, v, preferred_element_type=jnp.float32)

    @pl.when(i == NBLK - 1)
    def _fin():
        att = acc_scr[...] / s_scr[...][:, :1]
        pooled = jnp.zeros((S, D), jnp.float32)
        mask = _head_mask(S)
        for h in range(H):
            pooled = pooled + att[h * S:(h + 1) * S, :] * mask[h * S:(h + 1) * S, :]
        pooled_ref[...] = pooled


@jax.jit
def _tc2(u0, u1, ys, dinv_rep, wk, bk, wv, bv, seed, qw):
    bd = lambda: pl.BlockSpec((128, D), lambda i: (i, 0))
    full = lambda r: pl.BlockSpec((r, D), lambda i: (0, 0))
    return pl.pallas_call(
        _tc2_kernel,
        grid=(NBLK,),
        in_specs=[bd(), bd(), bd(), bd(),
                  full(D), full(1), full(D), full(1), full(S), full(D)],
        out_specs=pl.BlockSpec((S, D), lambda i: (0, 0)),
        out_shape=jax.ShapeDtypeStruct((S, D), jnp.float32),
        scratch_shapes=[pltpu.VMEM((H * S, D), jnp.float32),
                        pltpu.VMEM((H * S, 128), jnp.float32),
                        pltpu.VMEM((H * S, 128), jnp.float32),
                        pltpu.VMEM((H * S, D), jnp.float32)],
        compiler_params=pltpu.CompilerParams(
            dimension_semantics=("arbitrary",)),
    )(u0, u1, ys, dinv_rep, wk, bk.reshape(1, D), wv, bv.reshape(1, D),
      seed, qw)


def _ln(x, g, b):
    mu = jnp.mean(x, -1, keepdims=True)
    var = jnp.mean((x - mu) ** 2, -1, keepdims=True)
    return (x - mu) / jnp.sqrt(var + 1e-5) * g + b


def _tc3_kernel(pooled_ref, a1seed_ref, a1opw_ref, a1opb_ref,
                a1lnhg_ref, a1lnhb_ref, a1ffw_ref, a1ffb_ref,
                a1lnzg_ref, a1lnzb_ref,
                ipw_ref, ipb_ref, op2w_ref, op2b_ref,
                lnh2g_ref, lnh2b_ref, ff2w_ref, ff2b_ref,
                lnz2g_ref, lnz2b_ref,
                wk3_ref, bk3_ref, wv3_ref, bv3_ref,
                a3seed_ref, a3qw_ref, a3opw_ref, a3opb_ref,
                a3lnhg_ref, a3lnhb_ref, a3ffw_ref, a3ffb_ref,
                a3lnzg_ref, a3lnzb_ref, headw_ref, headb_ref,
                out_ref):
    dotT = lambda a, b: lax.dot_general(
        a, b, (((1,), (1,)), ((), ())), preferred_element_type=jnp.float32)

    out1 = dotT(pooled_ref[...], a1opw_ref[...]) + a1opb_ref[...]
    e = _ln(a1seed_ref[...] + out1, a1lnhg_ref[...], a1lnhb_ref[...])
    e = _ln(e + dotT(e, a1ffw_ref[...]) + a1ffb_ref[...],
            a1lnzg_ref[...], a1lnzb_ref[...])

    qkv = dotT(e, ipw_ref[...]) + ipb_ref[...]
    q2 = qkv[:, 0:D] * (DH ** -0.5)
    k2 = qkv[:, D:2 * D]
    v2 = qkv[:, 2 * D:3 * D]
    mask_s = _head_mask(S)
    q2big = jnp.tile(q2, (H, 1)) * mask_s
    logits2 = dotT(q2big, k2)
    a2 = jax.nn.softmax(logits2, axis=-1)
    att2 = jnp.dot(a2, v2, preferred_element_type=jnp.float32)
    out2 = jnp.zeros((S, D), jnp.float32)
    for h in range(H):
        out2 = out2 + att2[h * S:(h + 1) * S, :] * mask_s[h * S:(h + 1) * S, :]
    out2 = dotT(out2, op2w_ref[...]) + op2b_ref[...]
    e2 = _ln(e + out2, lnh2g_ref[...], lnh2b_ref[...])
    e2 = _ln(e2 + dotT(e2, ff2w_ref[...]) + ff2b_ref[...],
             lnz2g_ref[...], lnz2b_ref[...])

    k3 = jnp.dot(e2, wk3_ref[...], preferred_element_type=jnp.float32) + bk3_ref[...]
    v3 = jnp.dot(e2, wv3_ref[...], preferred_element_type=jnp.float32) + bv3_ref[...]
    q3 = dotT(a3seed_ref[...], a3qw_ref[...]) * (DH ** -0.5)
    mask_1 = _head_mask(1)
    q3big = jnp.tile(q3, (H, 1)) * mask_1
    logits3 = dotT(q3big, k3)
    a3 = jax.nn.softmax(logits3, axis=-1)
    att3 = jnp.dot(a3, v3, preferred_element_type=jnp.float32)
    out3 = jnp.sum(att3 * mask_1, axis=0, keepdims=True)
    out3 = dotT(out3, a3opw_ref[...]) + a3opb_ref[...]
    e3 = _ln(a3seed_ref[...] + out3, a3lnhg_ref[...], a3lnhb_ref[...])
    e3 = _ln(e3 + dotT(e3, a3ffw_ref[...]) + a3ffb_ref[...],
             a3lnzg_ref[...], a3lnzb_ref[...])
    out_ref[...] = jax.nn.sigmoid(
        jnp.dot(e3, headw_ref[...], preferred_element_type=jnp.float32)
        + headb_ref[...])


@jax.jit
def _tc3(pooled, p):
    cat = lambda w: w.transpose(1, 0, 2).reshape(D, D)
    row = lambda b: b.reshape(1, -1)
    args = [
        pooled,
        p["a1_seed"], p["a1_opW"], row(p["a1_opb"]),
        row(p["a1_lnhg"]), row(p["a1_lnhb"]), p["a1_ffW"], row(p["a1_ffb"]),
        row(p["a1_lnzg"]), row(p["a1_lnzb"]),
        p["in_projW"], row(p["in_projb"]), p["op2W"], row(p["op2b"]),
        row(p["lnh2g"]), row(p["lnh2b"]), p["ff2W"], row(p["ff2b"]),
        row(p["lnz2g"]), row(p["lnz2b"]),
        cat(p["a3_Wk"]), row(p["a3_bk"]), cat(p["a3_Wv"]), row(p["a3_bv"]),
        p["a3_seed"], p["a3_q"], p["a3_opW"], row(p["a3_opb"]),
        row(p["a3_lnhg"]), row(p["a3_lnhb"]), p["a3_ffW"], row(p["a3_ffb"]),
        row(p["a3_lnzg"]), row(p["a3_lnzb"]), p["headW"], row(p["headb"]),
    ]
    specs = [pl.BlockSpec(a.shape, lambda i: tuple(0 for _ in a.shape))
             for a in args]
    return pl.pallas_call(
        _tc3_kernel,
        grid=(1,),
        in_specs=specs,
        out_specs=pl.BlockSpec((1, D), lambda i: (0, 0)),
        out_shape=jax.ShapeDtypeStruct((1, D), jnp.float32),
    )(*args)


def kernel(x, edge_index, params):
    p = params
    src = edge_index[0]
    dst = edge_index[1]

    npad = EP - E
    pad_idx = (N + (jnp.arange(npad, dtype=jnp.int32) % SPARE))
    src_p = jnp.concatenate([src, pad_idx]).reshape(NW, NCHUNK, CH)
    dst_p = jnp.concatenate([dst, pad_idx]).reshape(NW, NCHUNK, CH)
    x_pad = jnp.pad(x, ((0, NP - N), (0, 0)))

    deg_parts = _deg_call(dst_p)
    dinv_rep, ys1 = _tc0(deg_parts, x_pad)

    u1 = _prop_call(ys1, src_p, dst_p)
    ys2 = _tc1(u1[0], u1[1], ys1, dinv_rep, p["conv1_W"], p["conv1_b"])
    u2 = _prop_call(ys2, src_p, dst_p)
    ys3 = _tc1(u2[0], u2[1], ys2, dinv_rep, p["conv2_W"], p["conv2_b"])
    u3 = _prop_call(ys3, src_p, dst_p)

    cat = lambda w: w.transpose(1, 0, 2).reshape(D, D)
    pooled = _tc2(u3[0], u3[1], ys3, dinv_rep,
                  cat(p["a1_Wk"]), p["a1_bk"].reshape(-1),
                  cat(p["a1_Wv"]), p["a1_bv"].reshape(-1),
                  p["a1_seed"], p["a1_q"])
    return _tc3(pooled, p)
